# baseline (speedup 1.0000x reference)
import functools

import jax
import jax.numpy as jnp
from jax import lax
from jax.experimental import pallas as pl
from jax.experimental.pallas import tpu as pltpu
from jax.experimental.pallas import tpu_sc as plsc

_L = 16
_C = 4096
_R = 4 * 4096
_NW = 32
_RB = 8
_ROWS_PER_W = _R // _NW
_N_CHUNKS = _ROWS_PER_W // _RB
_J = _C // _L


def _permute_body(y_hbm, perm_hbm, out_hbm, perm_v, in_v, out_v):
    wid = lax.axis_index("s") * 2 + lax.axis_index("c")
    pltpu.sync_copy(perm_hbm, perm_v)

    def chunk_body(cidx, _):
        base = (wid * _ROWS_PER_W + cidx * _RB) * _C
        pltpu.sync_copy(y_hbm.at[pl.ds(base, _RB * _C)], in_v)

        def row_body(r, _):
            roff = r * _C

            def col_body(j, _):
                idx = perm_v[pl.ds(j * _L, _L)] + roff
                out_v[pl.ds(roff + j * _L, _L)] = plsc.load_gather(
                    in_v, [idx])
                return 0

            return lax.fori_loop(0, _J, col_body, 0, unroll=4)

        lax.fori_loop(0, _RB, row_body, 0)
        pltpu.sync_copy(out_v, out_hbm.at[pl.ds(base, _RB * _C)])
        return 0

    lax.fori_loop(0, _N_CHUNKS, chunk_body, 0)


@jax.jit
def _permute(y_flat, perm_i32):
    mesh = plsc.VectorSubcoreMesh(core_axis_name="c", subcore_axis_name="s")
    f = functools.partial(
        pl.kernel,
        mesh=mesh,
        out_type=jax.ShapeDtypeStruct((_R * _C,), jnp.float32),
        scratch_types=[
            pltpu.VMEM((_C,), jnp.int32),
            pltpu.VMEM((_RB * _C,), jnp.float32),
            pltpu.VMEM((_RB * _C,), jnp.float32),
        ],
        compiler_params=pltpu.CompilerParams(needs_layout_passes=False),
    )(_permute_body)
    return f(y_flat, perm_i32)


def kernel(y, perm):
    out = _permute(y.reshape(-1), perm.astype(jnp.int32))
    return out.reshape(y.shape)


# speedup vs baseline: 1.2923x; 1.2923x over previous
import functools

import jax
import jax.numpy as jnp
from jax import lax
from jax.experimental import pallas as pl
from jax.experimental.pallas import tpu as pltpu
from jax.experimental.pallas import tpu_sc as plsc

_L = 16
_C = 4096
_R = 4 * 4096
_NW = 32
_RB = 4
_ROWS_PER_W = _R // _NW
_N_CHUNKS = _ROWS_PER_W // _RB
_J = _C // _L


def _permute_body(y_hbm, perm_hbm, out_hbm,
                  perm_v, in0, in1, out0, out1, si0, si1, so0, so1):
    wid = lax.axis_index("s") * 2 + lax.axis_index("c")
    row0 = wid * _ROWS_PER_W
    pltpu.sync_copy(perm_hbm, perm_v)

    ins = (in0, in1)
    outs = (out0, out1)
    sis = (si0, si1)
    sos = (so0, so1)

    def in_slice(c):
        return y_hbm.at[pl.ds(row0 + c * _RB, _RB)]

    def out_slice(c):
        return out_hbm.at[pl.ds(row0 + c * _RB, _RB)]

    pltpu.async_copy(in_slice(0), in0, si0)
    pltpu.async_copy(in_slice(1), in1, si1)

    @pl.loop(0, _N_CHUNKS, step=2)
    def chunk_loop(c0):
        for b in range(2):
            c = c0 + b
            in_v, out_v, si, so = ins[b], outs[b], sis[b], sos[b]
            pltpu.make_async_copy(in_slice(c), in_v, si).wait()
            @pl.when(c >= 2)
            def _():
                pltpu.make_async_copy(out_v, out_slice(c - 2), so).wait()

            def row_body(r, _):
                row_vec = jnp.full((_L,), r, jnp.int32)

                def col_body(j, _):
                    idx = perm_v[pl.ds(j * _L, _L)]
                    out_v[r, pl.ds(j * _L, _L)] = plsc.load_gather(
                        in_v, [row_vec, idx])
                    return 0

                lax.fori_loop(0, _J, col_body, 0, unroll=8)
                return 0

            lax.fori_loop(0, _RB, row_body, 0)
            pltpu.async_copy(out_v, out_slice(c), so)

            @pl.when(c + 2 < _N_CHUNKS)
            def _():
                pltpu.async_copy(in_slice(c + 2), in_v, si)

    pltpu.make_async_copy(out0, out_slice(_N_CHUNKS - 2), so0).wait()
    pltpu.make_async_copy(out1, out_slice(_N_CHUNKS - 1), so1).wait()


@jax.jit
def _permute(y2, perm_i32):
    mesh = plsc.VectorSubcoreMesh(core_axis_name="c", subcore_axis_name="s")
    f = functools.partial(
        pl.kernel,
        mesh=mesh,
        out_type=jax.ShapeDtypeStruct((_R, _C), jnp.float32),
        scratch_types=[
            pltpu.VMEM((_C,), jnp.int32),
            pltpu.VMEM((_RB, _C), jnp.float32),
            pltpu.VMEM((_RB, _C), jnp.float32),
            pltpu.VMEM((_RB, _C), jnp.float32),
            pltpu.VMEM((_RB, _C), jnp.float32),
            pltpu.SemaphoreType.DMA,
            pltpu.SemaphoreType.DMA,
            pltpu.SemaphoreType.DMA,
            pltpu.SemaphoreType.DMA,
        ],
        compiler_params=pltpu.CompilerParams(needs_layout_passes=False),
    )(_permute_body)
    return f(y2, perm_i32)


def kernel(y, perm):
    out = _permute(y.reshape(_R, _C), perm.astype(jnp.int32))
    return out.reshape(y.shape)


# speedup vs baseline: 8.5768x; 6.6369x over previous
import functools

import jax
import jax.numpy as jnp
from jax import lax
from jax.experimental import pallas as pl
from jax.experimental.pallas import tpu as pltpu
from jax.experimental.pallas import tpu_sc as plsc

_L = 16
_C = 4096
_R = 4 * 4096
_NW = 32
_RB = 4
_ROWS_PER_W = _R // _NW
_N_CHUNKS = _ROWS_PER_W // _RB
_J = _C // _L
_V = _RB * _J


def _permute_body(y_hbm, perm_hbm, out_hbm,
                  perm_v, idx_v, in0, in1, out0, out1, si0, si1, so0, so1):
    wid = lax.axis_index("s") * 2 + lax.axis_index("c")
    row0 = wid * _ROWS_PER_W
    pltpu.sync_copy(perm_hbm, perm_v)

    @plsc.parallel_loop(0, _V, unroll=8)
    def build_idx(i):
        r = i // _J
        j = i - r * _J
        idx_v[pl.ds(i * _L, _L)] = perm_v[pl.ds(j * _L, _L)] + r * _C

    ins = (in0, in1)
    outs = (out0, out1)
    sis = (si0, si1)
    sos = (so0, so1)

    def load_chunk(c, in_v, si):
        for r in range(_RB):
            pltpu.async_copy(y_hbm.at[row0 + c * _RB + r],
                             in_v.at[pl.ds(r * _C, _C)], si)

    def store_chunk(c, out_v, so):
        for r in range(_RB):
            pltpu.async_copy(out_v.at[pl.ds(r * _C, _C)],
                             out_hbm.at[row0 + c * _RB + r], so)

    def wait_load(c, in_v, si):
        for r in range(_RB):
            pltpu.make_async_copy(y_hbm.at[row0 + c * _RB + r],
                                  in_v.at[pl.ds(r * _C, _C)], si).wait()

    def wait_store(c, out_v, so):
        for r in range(_RB):
            pltpu.make_async_copy(out_v.at[pl.ds(r * _C, _C)],
                                  out_hbm.at[row0 + c * _RB + r], so).wait()

    load_chunk(0, in0, si0)
    load_chunk(1, in1, si1)

    @pl.loop(0, _N_CHUNKS, step=2)
    def chunk_loop(c0):
        for b in range(2):
            c = c0 + b
            in_v, out_v, si, so = ins[b], outs[b], sis[b], sos[b]
            wait_load(c, in_v, si)
            @pl.when(c >= 2)
            def _():
                wait_store(c - 2, out_v, so)

            @plsc.parallel_loop(0, _V, unroll=8)
            def gather(i):
                idx = idx_v[pl.ds(i * _L, _L)]
                out_v[pl.ds(i * _L, _L)] = plsc.load_gather(in_v, [idx])

            store_chunk(c, out_v, so)

            @pl.when(c + 2 < _N_CHUNKS)
            def _():
                load_chunk(c + 2, in_v, si)

    wait_store(_N_CHUNKS - 2, out0, so0)
    wait_store(_N_CHUNKS - 1, out1, so1)


@jax.jit
def _permute(y2, perm_i32):
    mesh = plsc.VectorSubcoreMesh(core_axis_name="c", subcore_axis_name="s")
    f = functools.partial(
        pl.kernel,
        mesh=mesh,
        out_type=jax.ShapeDtypeStruct((_R, _C), jnp.float32),
        scratch_types=[
            pltpu.VMEM((_C,), jnp.int32),
            pltpu.VMEM((_RB * _C,), jnp.int32),
            pltpu.VMEM((_RB * _C,), jnp.float32),
            pltpu.VMEM((_RB * _C,), jnp.float32),
            pltpu.VMEM((_RB * _C,), jnp.float32),
            pltpu.VMEM((_RB * _C,), jnp.float32),
            pltpu.SemaphoreType.DMA,
            pltpu.SemaphoreType.DMA,
            pltpu.SemaphoreType.DMA,
            pltpu.SemaphoreType.DMA,
        ],
        compiler_params=pltpu.CompilerParams(needs_layout_passes=False),
    )(_permute_body)
    return f(y2, perm_i32)


def kernel(y, perm):
    out = _permute(y.reshape(_R, _C), perm.astype(jnp.int32))
    return out.reshape(y.shape)


# speedup vs baseline: 8.6369x; 1.0070x over previous
import functools

import jax
import jax.numpy as jnp
from jax import lax
from jax.experimental import pallas as pl
from jax.experimental.pallas import tpu as pltpu
from jax.experimental.pallas import tpu_sc as plsc

_L = 16
_C = 4096
_R = 4 * 4096
_NW = 32
_RB = 4
_ROWS_PER_W = _R // _NW
_N_CHUNKS = _ROWS_PER_W // _RB
_J = _C // _L
_V = _RB * _J


def _permute_body(y_hbm, perm_hbm, out_hbm,
                  perm_v, idx_v, in0, in1, out0, out1, si0, si1, so0, so1):
    wid = lax.axis_index("s") * 2 + lax.axis_index("c")
    row0 = wid * _ROWS_PER_W
    pltpu.sync_copy(perm_hbm, perm_v)

    @plsc.parallel_loop(0, _V, unroll=8)
    def build_idx(i):
        r = i // _J
        j = i - r * _J
        idx_v[pl.ds(i * _L, _L)] = perm_v[pl.ds(j * _L, _L)] + r * _C

    ins = (in0, in1)
    outs = (out0, out1)
    sis = (si0, si1)
    sos = (so0, so1)

    def load_chunk(c, in_v, si):
        for r in range(_RB):
            pltpu.async_copy(y_hbm.at[row0 + c * _RB + r],
                             in_v.at[pl.ds(r * _C, _C)], si)

    def store_chunk(c, out_v, so):
        for r in range(_RB):
            pltpu.async_copy(out_v.at[pl.ds(r * _C, _C)],
                             out_hbm.at[row0 + c * _RB + r], so)

    def wait_load(c, in_v, si):
        for r in range(_RB):
            pltpu.make_async_copy(y_hbm.at[row0 + c * _RB + r],
                                  in_v.at[pl.ds(r * _C, _C)], si).wait()

    def wait_store(c, out_v, so):
        for r in range(_RB):
            pltpu.make_async_copy(out_v.at[pl.ds(r * _C, _C)],
                                  out_hbm.at[row0 + c * _RB + r], so).wait()

    load_chunk(0, in0, si0)
    load_chunk(1, in1, si1)

    @pl.loop(0, _N_CHUNKS, step=2)
    def chunk_loop(c0):
        for b in range(2):
            c = c0 + b
            in_v, out_v, si, so = ins[b], outs[b], sis[b], sos[b]
            wait_load(c, in_v, si)
            @pl.when(c >= 2)
            def _():
                wait_store(c - 2, out_v, so)

            @plsc.parallel_loop(0, _V, unroll=16)
            def gather(i):
                idx = idx_v[pl.ds(i * _L, _L)]
                out_v[pl.ds(i * _L, _L)] = plsc.load_gather(in_v, [idx])

            store_chunk(c, out_v, so)

            @pl.when(c + 2 < _N_CHUNKS)
            def _():
                load_chunk(c + 2, in_v, si)

    wait_store(_N_CHUNKS - 2, out0, so0)
    wait_store(_N_CHUNKS - 1, out1, so1)


@jax.jit
def _permute(y2, perm_i32):
    mesh = plsc.VectorSubcoreMesh(core_axis_name="c", subcore_axis_name="s")
    f = functools.partial(
        pl.kernel,
        mesh=mesh,
        out_type=jax.ShapeDtypeStruct((_R, _C), jnp.float32),
        scratch_types=[
            pltpu.VMEM((_C,), jnp.int32),
            pltpu.VMEM((_RB * _C,), jnp.int32),
            pltpu.VMEM((_RB * _C,), jnp.float32),
            pltpu.VMEM((_RB * _C,), jnp.float32),
            pltpu.VMEM((_RB * _C,), jnp.float32),
            pltpu.VMEM((_RB * _C,), jnp.float32),
            pltpu.SemaphoreType.DMA,
            pltpu.SemaphoreType.DMA,
            pltpu.SemaphoreType.DMA,
            pltpu.SemaphoreType.DMA,
        ],
        compiler_params=pltpu.CompilerParams(needs_layout_passes=False),
    )(_permute_body)
    return f(y2, perm_i32)


def kernel(y, perm):
    out = _permute(y.reshape(_R, _C), perm.astype(jnp.int32))
    return out.reshape(y.shape)


# speedup vs baseline: 9.4991x; 1.0998x over previous
import functools

import jax
import jax.numpy as jnp
from jax import lax
from jax.experimental import pallas as pl
from jax.experimental.pallas import tpu as pltpu
from jax.experimental.pallas import tpu_sc as plsc

_L = 16
_C = 4096
_R = 4 * 4096
_NW = 32
_RB = 4
_ROWS_PER_W = _R // _NW
_N_CHUNKS = _ROWS_PER_W // _RB
_J = _C // _L
_V = _RB * _J


def _permute_body(y_hbm, perm_hbm, out_hbm,
                  perm_v, in0, in1, out0, out1, si0, si1, so0, so1):
    wid = lax.axis_index("s") * 2 + lax.axis_index("c")
    row0 = wid * _ROWS_PER_W
    pltpu.sync_copy(perm_hbm, perm_v)

    ins = (in0, in1)
    outs = (out0, out1)
    sis = (si0, si1)
    sos = (so0, so1)

    def load_chunk(c, in_v, si):
        for r in range(_RB):
            pltpu.async_copy(y_hbm.at[row0 + c * _RB + r],
                             in_v.at[pl.ds(r * _C, _C)], si)

    def store_chunk(c, out_v, so):
        for r in range(_RB):
            pltpu.async_copy(out_v.at[pl.ds(r * _C, _C)],
                             out_hbm.at[row0 + c * _RB + r], so)

    def wait_load(c, in_v, si):
        for r in range(_RB):
            pltpu.make_async_copy(y_hbm.at[row0 + c * _RB + r],
                                  in_v.at[pl.ds(r * _C, _C)], si).wait()

    def wait_store(c, out_v, so):
        for r in range(_RB):
            pltpu.make_async_copy(out_v.at[pl.ds(r * _C, _C)],
                                  out_hbm.at[row0 + c * _RB + r], so).wait()

    load_chunk(0, in0, si0)
    load_chunk(1, in1, si1)

    @pl.loop(0, _N_CHUNKS, step=2)
    def chunk_loop(c0):
        for b in range(2):
            c = c0 + b
            in_v, out_v, si, so = ins[b], outs[b], sis[b], sos[b]
            wait_load(c, in_v, si)
            @pl.when(c >= 2)
            def _():
                wait_store(c - 2, out_v, so)

            @plsc.parallel_loop(0, _J, unroll=4)
            def gather(j):
                pj = perm_v[pl.ds(j * _L, _L)]
                for r in range(_RB):
                    out_v[pl.ds(r * _C + j * _L, _L)] = plsc.load_gather(
                        in_v, [pj + r * _C])

            store_chunk(c, out_v, so)

            @pl.when(c + 2 < _N_CHUNKS)
            def _():
                load_chunk(c + 2, in_v, si)

    wait_store(_N_CHUNKS - 2, out0, so0)
    wait_store(_N_CHUNKS - 1, out1, so1)


@jax.jit
def _permute(y2, perm_i32):
    mesh = plsc.VectorSubcoreMesh(core_axis_name="c", subcore_axis_name="s")
    f = functools.partial(
        pl.kernel,
        mesh=mesh,
        out_type=jax.ShapeDtypeStruct((_R, _C), jnp.float32),
        scratch_types=[
            pltpu.VMEM((_C,), jnp.int32),
            pltpu.VMEM((_RB * _C,), jnp.float32),
            pltpu.VMEM((_RB * _C,), jnp.float32),
            pltpu.VMEM((_RB * _C,), jnp.float32),
            pltpu.VMEM((_RB * _C,), jnp.float32),
            pltpu.SemaphoreType.DMA,
            pltpu.SemaphoreType.DMA,
            pltpu.SemaphoreType.DMA,
            pltpu.SemaphoreType.DMA,
        ],
        compiler_params=pltpu.CompilerParams(needs_layout_passes=False),
    )(_permute_body)
    return f(y2, perm_i32)


def kernel(y, perm):
    out = _permute(y.reshape(_R, _C), perm.astype(jnp.int32))
    return out.reshape(y.shape)


# speedup vs baseline: 9.5260x; 1.0028x over previous
import functools

import jax
import jax.numpy as jnp
from jax import lax
from jax.experimental import pallas as pl
from jax.experimental.pallas import tpu as pltpu
from jax.experimental.pallas import tpu_sc as plsc

_L = 16
_C = 4096
_R = 4 * 4096
_NW = 32
_RB = 4
_ROWS_PER_W = _R // _NW
_N_CHUNKS = _ROWS_PER_W // _RB
_J = _C // _L


def _permute_body(y_hbm, perm_hbm, out_hbm,
                  perm_v, in0, in1, out0, out1, si0, si1, so0, so1):
    wid = lax.axis_index("s") * 2 + lax.axis_index("c")
    row0 = wid * _ROWS_PER_W
    pltpu.sync_copy(perm_hbm, perm_v)

    rvecs = [jnp.full((_L,), r, jnp.int32) for r in range(_RB)]

    ins = (in0, in1)
    outs = (out0, out1)
    sis = (si0, si1)
    sos = (so0, so1)

    def in_slice(c):
        return y_hbm.at[pl.ds(row0 + c * _RB, _RB)]

    def out_slice(c):
        return out_hbm.at[pl.ds(row0 + c * _RB, _RB)]

    pltpu.async_copy(in_slice(0), in0, si0)
    pltpu.async_copy(in_slice(1), in1, si1)

    @pl.loop(0, _N_CHUNKS, step=2)
    def chunk_loop(c0):
        for b in range(2):
            c = c0 + b
            in_v, out_v, si, so = ins[b], outs[b], sis[b], sos[b]
            pltpu.make_async_copy(in_slice(c), in_v, si).wait()
            @pl.when(c >= 2)
            def _():
                pltpu.make_async_copy(out_v, out_slice(c - 2), so).wait()

            @plsc.parallel_loop(0, _J, unroll=4)
            def gather(j):
                pj = perm_v[pl.ds(j * _L, _L)]
                for r in range(_RB):
                    out_v[r, pl.ds(j * _L, _L)] = plsc.load_gather(
                        in_v, [rvecs[r], pj])

            pltpu.async_copy(out_v, out_slice(c), so)

            @pl.when(c + 2 < _N_CHUNKS)
            def _():
                pltpu.async_copy(in_slice(c + 2), in_v, si)

    pltpu.make_async_copy(out0, out_slice(_N_CHUNKS - 2), so0).wait()
    pltpu.make_async_copy(out1, out_slice(_N_CHUNKS - 1), so1).wait()


@jax.jit
def _permute(y2, perm_i32):
    mesh = plsc.VectorSubcoreMesh(core_axis_name="c", subcore_axis_name="s")
    f = functools.partial(
        pl.kernel,
        mesh=mesh,
        out_type=jax.ShapeDtypeStruct((_R, _C), jnp.float32),
        scratch_types=[
            pltpu.VMEM((_C,), jnp.int32),
            pltpu.VMEM((_RB, _C), jnp.float32),
            pltpu.VMEM((_RB, _C), jnp.float32),
            pltpu.VMEM((_RB, _C), jnp.float32),
            pltpu.VMEM((_RB, _C), jnp.float32),
            pltpu.SemaphoreType.DMA,
            pltpu.SemaphoreType.DMA,
            pltpu.SemaphoreType.DMA,
            pltpu.SemaphoreType.DMA,
        ],
        compiler_params=pltpu.CompilerParams(needs_layout_passes=False),
    )(_permute_body)
    return f(y2, perm_i32)


def kernel(y, perm):
    out = _permute(y.reshape(_R, _C), perm.astype(jnp.int32))
    return out.reshape(y.shape)


# speedup vs baseline: 9.7867x; 1.0274x over previous
import functools

import jax
import jax.numpy as jnp
from jax import lax
from jax.experimental import pallas as pl
from jax.experimental.pallas import tpu as pltpu
from jax.experimental.pallas import tpu_sc as plsc

_L = 16
_C = 4096
_R = 4 * 4096
_NW = 32
_RB = 4
_NBI = 4
_NBO = 2
_ROWS_PER_W = _R // _NW
_N_CHUNKS = _ROWS_PER_W // _RB
_J = _C // _L


def _permute_body(y_hbm, perm_hbm, out_hbm, perm_v,
                  in0, in1, in2, in3, out0, out1,
                  si0, si1, si2, si3, so0, so1):
    wid = lax.axis_index("s") * 2 + lax.axis_index("c")
    row0 = wid * _ROWS_PER_W
    pltpu.sync_copy(perm_hbm, perm_v)

    rvecs = [jnp.full((_L,), r, jnp.int32) for r in range(_RB)]

    ins = (in0, in1, in2, in3)
    outs = (out0, out1)
    sis = (si0, si1, si2, si3)
    sos = (so0, so1)

    def in_slice(c):
        return y_hbm.at[pl.ds(row0 + c * _RB, _RB)]

    def out_slice(c):
        return out_hbm.at[pl.ds(row0 + c * _RB, _RB)]

    for b in range(_NBI):
        pltpu.async_copy(in_slice(b), ins[b], sis[b])

    @pl.loop(0, _N_CHUNKS, step=_NBI)
    def chunk_loop(c0):
        for b in range(_NBI):
            c = c0 + b
            in_v, si = ins[b], sis[b]
            out_v, so = outs[b % _NBO], sos[b % _NBO]
            pltpu.make_async_copy(in_slice(c), in_v, si).wait()
            @pl.when(c >= _NBO)
            def _():
                pltpu.make_async_copy(out_v, out_slice(c - _NBO), so).wait()

            @plsc.parallel_loop(0, _J, unroll=8)
            def gather(j):
                pj = perm_v[pl.ds(j * _L, _L)]
                for r in range(_RB):
                    out_v[r, pl.ds(j * _L, _L)] = plsc.load_gather(
                        in_v, [rvecs[r], pj])

            pltpu.async_copy(out_v, out_slice(c), so)

            @pl.when(c + _NBI < _N_CHUNKS)
            def _():
                pltpu.async_copy(in_slice(c + _NBI), in_v, si)

    pltpu.make_async_copy(out0, out_slice(_N_CHUNKS - 2), so0).wait()
    pltpu.make_async_copy(out1, out_slice(_N_CHUNKS - 1), so1).wait()


@jax.jit
def _permute(y2, perm_i32):
    mesh = plsc.VectorSubcoreMesh(core_axis_name="c", subcore_axis_name="s")
    f = functools.partial(
        pl.kernel,
        mesh=mesh,
        out_type=jax.ShapeDtypeStruct((_R, _C), jnp.float32),
        scratch_types=(
            [pltpu.VMEM((_C,), jnp.int32)]
            + [pltpu.VMEM((_RB, _C), jnp.float32)] * (_NBI + _NBO)
            + [pltpu.SemaphoreType.DMA] * (_NBI + _NBO)
        ),
        compiler_params=pltpu.CompilerParams(needs_layout_passes=False),
    )(_permute_body)
    return f(y2, perm_i32)


def kernel(y, perm):
    out = _permute(y.reshape(_R, _C), perm.astype(jnp.int32))
    return out.reshape(y.shape)


# speedup vs baseline: 9.7929x; 1.0006x over previous
import functools

import jax
import jax.numpy as jnp
from jax import lax
from jax.experimental import pallas as pl
from jax.experimental.pallas import tpu as pltpu
from jax.experimental.pallas import tpu_sc as plsc

_L = 16
_C = 4096
_R = 4 * 4096
_NW = 32
_RB = 4
_NBI = 4
_NBO = 2
_ROWS_PER_W = _R // _NW
_N_CHUNKS = _ROWS_PER_W // _RB
_J = _C // _L


def _permute_body(y_hbm, perm_hbm, out_hbm, perm_v,
                  in0, in1, in2, in3, out0, out1,
                  si0, si1, si2, si3, so0, so1):
    wid = lax.axis_index("s") * 2 + lax.axis_index("c")
    row0 = wid * _ROWS_PER_W
    pltpu.sync_copy(perm_hbm, perm_v)

    rvecs = [jnp.full((_L,), r, jnp.int32) for r in range(_RB)]

    ins = (in0, in1, in2, in3)
    outs = (out0, out1)
    sis = (si0, si1, si2, si3)
    sos = (so0, so1)

    def in_slice(c):
        return y_hbm.at[pl.ds(row0 + c * _RB, _RB)]

    def out_slice(c):
        return out_hbm.at[pl.ds(row0 + c * _RB, _RB)]

    for b in range(_NBI):
        pltpu.async_copy(in_slice(b), ins[b], sis[b])

    @pl.loop(0, _N_CHUNKS, step=_NBI)
    def chunk_loop(c0):
        for b in range(_NBI):
            c = c0 + b
            in_v, si = ins[b], sis[b]
            out_v, so = outs[b % _NBO], sos[b % _NBO]
            pltpu.make_async_copy(in_slice(c), in_v, si).wait()
            @pl.when(c >= _NBO)
            def _():
                pltpu.make_async_copy(out_v, out_slice(c - _NBO), so).wait()

            @plsc.parallel_loop(0, _J, unroll=4)
            def gather(j):
                pj = perm_v[pl.ds(j * _L, _L)]
                for r in range(_RB):
                    out_v[r, pl.ds(j * _L, _L)] = plsc.load_gather(
                        in_v, [rvecs[r], pj])

            pltpu.async_copy(out_v, out_slice(c), so)

            @pl.when(c + _NBI < _N_CHUNKS)
            def _():
                pltpu.async_copy(in_slice(c + _NBI), in_v, si)

    pltpu.make_async_copy(out0, out_slice(_N_CHUNKS - 2), so0).wait()
    pltpu.make_async_copy(out1, out_slice(_N_CHUNKS - 1), so1).wait()


@jax.jit
def _permute(y2, perm_i32):
    mesh = plsc.VectorSubcoreMesh(core_axis_name="c", subcore_axis_name="s")
    f = functools.partial(
        pl.kernel,
        mesh=mesh,
        out_type=jax.ShapeDtypeStruct((_R, _C), jnp.float32),
        scratch_types=(
            [pltpu.VMEM((_C,), jnp.int32)]
            + [pltpu.VMEM((_RB, _C), jnp.float32)] * (_NBI + _NBO)
            + [pltpu.SemaphoreType.DMA] * (_NBI + _NBO)
        ),
        compiler_params=pltpu.CompilerParams(needs_layout_passes=False),
    )(_permute_body)
    return f(y2, perm_i32)


def kernel(y, perm):
    out = _permute(y.reshape(_R, _C), perm.astype(jnp.int32))
    return out.reshape(y.shape)
